# Initial kernel scaffold; baseline (speedup 1.0000x reference)
import jax
import jax.numpy as jnp
from jax.experimental import pallas as pl


def kernel(x, skip_w_0, skip_b_0, w1_khat_0, g1_0, beta1_0, w2_khat_0, g2_0, beta2_0, skip_w_1, skip_b_1, w1_khat_1, g1_1, beta1_1, w2_khat_1, g2_1, beta2_1, skip_w_2, skip_b_2, w1_khat_2, g1_2, beta1_2, w2_khat_2, g2_2, beta2_2):
    raise NotImplementedError("write your pallas kernel here")


# speedup vs baseline: 4.6445x; 4.6445x over previous
import functools

import numpy as np

import jax
import jax.numpy as jnp
from jax.experimental import pallas as pl
from jax.experimental.pallas import tpu as pltpu

_BN_EPS = 1e-5
_LANE = 128
_SHIFTS = tuple((s1, s2) for s1 in (-1, 0, 1) for s2 in (-1, 0, 1))


def _round_up(n, m):
    return ((n + m - 1) // m) * m


@functools.lru_cache(maxsize=4)
def _tap_matrix_np(h, w):
    wf = w // 2 + 1
    f = h * wf
    fp = _round_up(f, _LANE)
    k = np.arange(f)
    k1 = k // wf
    k2 = k % wf
    c = np.where((k2 == 0) | ((w % 2 == 0) & (k2 == wf - 1)), 1.0, 2.0)
    c = c / float(h * w)
    m = np.zeros((2 * fp, 16), np.float32)
    for t, (s1, s2) in enumerate(_SHIFTS):
        th = 2.0 * np.pi * (k1 * float(s1 % h) / h + k2 * float(s2 % w) / w)
        m[:f, t] = c * np.cos(th)
        m[fp:fp + f, t] = -c * np.sin(th)
    return m


def _tap_extract_kernel(w_ref, mhi_ref, mlo_ref, o_ref):
    w = w_ref[...]
    w_hi = w.astype(jnp.bfloat16)
    w_lo = (w - w_hi.astype(jnp.float32)).astype(jnp.bfloat16)
    m_hi = mhi_ref[...]
    m_lo = mlo_ref[...]
    o_ref[...] = (
        jnp.dot(w_hi, m_hi, preferred_element_type=jnp.float32)
        + jnp.dot(w_hi, m_lo, preferred_element_type=jnp.float32)
        + jnp.dot(w_lo, m_hi, preferred_element_type=jnp.float32))


def _extract_taps(w_khat, h, w):
    co, ci, two_fp = w_khat.shape
    r = co * ci
    w2 = w_khat.reshape(r, two_fp)
    rt = r if r <= 1024 else 1024
    grid = (r // rt,)
    m_np = _tap_matrix_np(h, w)
    m_hi = jnp.asarray(m_np).astype(jnp.bfloat16)
    m_lo = jnp.asarray(m_np - m_hi.astype(jnp.float32)).astype(jnp.bfloat16)
    taps16 = pl.pallas_call(
        _tap_extract_kernel,
        out_shape=jax.ShapeDtypeStruct((r, 16), jnp.float32),
        grid=grid,
        in_specs=[pl.BlockSpec((rt, two_fp), lambda i: (i, 0)),
                  pl.BlockSpec((two_fp, 16), lambda i: (0, 0)),
                  pl.BlockSpec((two_fp, 16), lambda i: (0, 0))],
        out_specs=pl.BlockSpec((rt, 16), lambda i: (i, 0)),
        compiler_params=pltpu.CompilerParams(
            dimension_semantics=("parallel",)),
        cost_estimate=pl.CostEstimate(
            flops=2 * r * two_fp * 16, transcendentals=0,
            bytes_accessed=4 * (r * two_fp + r * 16)),
    )(w2, m_hi, m_lo)
    taps = taps16[:, :9].reshape(co, ci, 9)
    return jnp.transpose(taps, (2, 1, 0))


def _roll(x, s, axis):
    n = x.shape[axis]
    k = (-s) % n
    if k == 0:
        return x
    lo = [slice(None)] * x.ndim
    hi = [slice(None)] * x.ndim
    lo[axis] = slice(k, None)
    hi[axis] = slice(None, k)
    return jnp.concatenate([x[tuple(lo)], x[tuple(hi)]], axis=axis)


def _roll2(x, s1, s2):
    return _roll(_roll(x, s1, 1), s2, 2)


def _conv3x3_acc(x, t_ref):
    b, h, w, ci = x.shape
    co = t_ref.shape[-1]
    n = b * h * w
    if ci == 1:
        acc = jnp.zeros((b, h, w, co), jnp.float32)
        for t, (s1, s2) in enumerate(_SHIFTS):
            xs = _roll2(x, s1, s2)
            acc = acc + xs * t_ref[t][0]
        return acc.reshape(n, co)
    acc = jnp.zeros((n, co), jnp.float32)
    for t, (s1, s2) in enumerate(_SHIFTS):
        xs = _roll2(x, s1, s2).reshape(n, ci)
        acc = acc + jnp.dot(xs, t_ref[t], preferred_element_type=jnp.float32)
    return acc


def _bn_scale_shift(y, g, bt, inv_n):
    s1 = jnp.sum(y, axis=0, keepdims=True)
    s2 = jnp.sum(y * y, axis=0, keepdims=True)
    mean = s1 * inv_n
    var = s2 * inv_n - mean * mean
    scale = g * jax.lax.rsqrt(var + _BN_EPS)
    shift = bt - mean * scale
    return scale, shift


def _conv_bn_relu_kernel(x_ref, t_ref, g_ref, b_ref, o_ref, *, inv_n):
    b, h, w, _ = x_ref.shape
    y = _conv3x3_acc(x_ref[...], t_ref)
    scale, shift = _bn_scale_shift(y, g_ref[...], b_ref[...], inv_n)
    z = jnp.maximum(y * scale + shift, 0.0)
    o_ref[...] = z.reshape(b, h, w, -1)


def _conv_bn_skip_relu_kernel(x_ref, t_ref, xin_ref, sw_ref, sb_ref,
                              g_ref, b_ref, o_ref, *, inv_n):
    b, h, w, _ = x_ref.shape
    n = b * h * w
    y = _conv3x3_acc(x_ref[...], t_ref)
    ci0 = xin_ref.shape[-1]
    if ci0 == 1:
        skip = xin_ref[...].reshape(n, 1) * sw_ref[...] + sb_ref[...]
    else:
        skip = jnp.dot(xin_ref[...].reshape(n, ci0), sw_ref[...],
                       preferred_element_type=jnp.float32) + sb_ref[...]
    scale, shift = _bn_scale_shift(y, g_ref[...], b_ref[...], inv_n)
    z = jnp.maximum(y * scale + shift + skip, 0.0)
    o_ref[...] = z.reshape(b, h, w, -1)


def _conv_bn_layer(x, taps, g, bt, xin=None, sw=None, sb=None):
    b, h, w, ci = x.shape
    co = taps.shape[-1]
    n = b * h * w
    inv_n = 1.0 / float(n)
    out_shape = jax.ShapeDtypeStruct((b, h, w, co), jnp.float32)
    g2 = g.reshape(1, co).astype(jnp.float32)
    bt2 = bt.reshape(1, co).astype(jnp.float32)
    flops = 2 * n * 9 * ci * co + 8 * n * co
    if xin is None:
        return pl.pallas_call(
            functools.partial(_conv_bn_relu_kernel, inv_n=inv_n),
            out_shape=out_shape,
            cost_estimate=pl.CostEstimate(
                flops=flops, transcendentals=co,
                bytes_accessed=4 * (x.size + taps.size + n * co)),
        )(x, taps, g2, bt2)
    ci0 = xin.shape[-1]
    return pl.pallas_call(
        functools.partial(_conv_bn_skip_relu_kernel, inv_n=inv_n),
        out_shape=out_shape,
        cost_estimate=pl.CostEstimate(
            flops=flops + 2 * n * ci0 * co, transcendentals=co,
            bytes_accessed=4 * (x.size + xin.size + taps.size + n * co)),
    )(x, taps, xin, sw, sb, g2, bt2)


def _res_block(xcl, skip_w, skip_b, w1_khat, g1, beta1, w2_khat, g2, beta2,
               h, w):
    co = w1_khat.shape[0]
    t1 = _extract_taps(w1_khat, h, w)
    t2 = _extract_taps(w2_khat, h, w)
    sw = jnp.transpose(skip_w[:, :, 0, 0], (1, 0)).astype(jnp.float32)
    sb = skip_b.reshape(1, co).astype(jnp.float32)
    z = _conv_bn_layer(xcl, t1, g1, beta1)
    return _conv_bn_layer(z, t2, g2, beta2, xin=xcl, sw=sw, sb=sb)


def kernel(x,
           skip_w_0, skip_b_0, w1_khat_0, g1_0, beta1_0, w2_khat_0, g2_0,
           beta2_0,
           skip_w_1, skip_b_1, w1_khat_1, g1_1, beta1_1, w2_khat_1, g2_1,
           beta2_1,
           skip_w_2, skip_b_2, w1_khat_2, g1_2, beta1_2, w2_khat_2, g2_2,
           beta2_2):
    b, _, h, w = x.shape
    xcl = jnp.transpose(x, (0, 2, 3, 1))
    groups = [
        (skip_w_0, skip_b_0, w1_khat_0, g1_0, beta1_0, w2_khat_0, g2_0,
         beta2_0),
        (skip_w_1, skip_b_1, w1_khat_1, g1_1, beta1_1, w2_khat_1, g2_1,
         beta2_1),
        (skip_w_2, skip_b_2, w1_khat_2, g1_2, beta1_2, w2_khat_2, g2_2,
         beta2_2),
    ]
    for blk in groups:
        xcl = _res_block(xcl, *blk, h, w)
    return jnp.transpose(xcl, (0, 3, 1, 2))
